# Initial kernel scaffold; baseline (speedup 1.0000x reference)
import jax
import jax.numpy as jnp
from jax.experimental import pallas as pl


def kernel(features, class_centers, representatives, W1, b1, W2, b2):
    raise NotImplementedError("write your pallas kernel here")


# speedup vs baseline: 86.9067x; 86.9067x over previous
import functools

import jax
import jax.numpy as jnp
from jax.experimental import pallas as pl
from jax.experimental.pallas import tpu as pltpu

N_REPS = 100000
N_CENTERS = 1000
TILE = 2048
N_TILES = 49
REPS_PAD = N_TILES * TILE
NEG = -jnp.inf


def _rownorm_scale(x, eps=1e-12):
    n = jnp.sqrt(jnp.sum(x * x, axis=1, keepdims=True))
    return x / jnp.maximum(n, eps)


def _extract_top5(x):
    r, w = x.shape
    ii = jax.lax.broadcasted_iota(jnp.int32, (r, w), 1)
    outs = []
    for _ in range(5):
        m = jnp.max(x, axis=1, keepdims=True)
        eq = x == m
        first = jnp.min(jnp.where(eq, ii, w), axis=1, keepdims=True)
        x = jnp.where(ii == first, NEG, x)
        outs.append(m)
    pad = jnp.full((r, 1), NEG, dtype=x.dtype)
    return jnp.concatenate(outs + [pad, pad, pad], axis=1), x


def _kernel(feat_ref, cent_ref, reps_ref, w1_ref, b1_ref, w2_ref, b2_ref,
            stats_ref, scores_ref, top_ref, sum_ref):
    i = pl.program_id(0)

    @pl.when(i == 0)
    def _init():
        top_ref[...] = jnp.full_like(top_ref, NEG)
        sum_ref[...] = jnp.zeros_like(sum_ref)

    feats = feat_ref[...]
    nf = _rownorm_scale(feats)

    reps = _rownorm_scale(reps_ref[...])
    sims = jax.lax.dot_general(
        nf, reps, (((1,), (1,)), ((), ())),
        preferred_element_type=jnp.float32)

    sum_ref[...] += jnp.sum(sims, axis=1, keepdims=True)

    gcol = jax.lax.broadcasted_iota(jnp.int32, sims.shape, 1) + i * TILE
    sims = jnp.where(gcol < N_REPS, sims, NEG)

    tile_top, _ = _extract_top5(sims)
    merged = jnp.concatenate([top_ref[...], tile_top], axis=1)
    new_top, _ = _extract_top5(merged)
    top_ref[...] = new_top

    @pl.when(i == N_TILES - 1)
    def _finish():
        cents = _rownorm_scale(cent_ref[...])
        csims = jax.lax.dot_general(
            nf, cents, (((1,), (1,)), ((), ())),
            preferred_element_type=jnp.float32)
        ccol = jax.lax.broadcasted_iota(jnp.int32, csims.shape, 1)
        max_center = jnp.max(jnp.where(ccol < N_CENTERS, csims, NEG),
                             axis=1, keepdims=True)
        mean_center = jnp.sum(csims, axis=1, keepdims=True) / N_CENTERS

        top = top_ref[...]
        rep_max = top[:, 0:1]
        rep_mean = (top[:, 0:1] + top[:, 1:2] + top[:, 2:3]
                    + top[:, 3:4] + top[:, 4:5]) * 0.2
        rep_gmean = sum_ref[...] / N_REPS
        feat_norm = jnp.sqrt(jnp.sum(feats * feats, axis=1, keepdims=True))

        zero = jnp.zeros_like(rep_max)
        stats8 = jnp.concatenate(
            [max_center, mean_center, rep_max, rep_mean, rep_gmean,
             feat_norm, zero, zero], axis=1)

        base = (max_center + mean_center + rep_max + rep_mean) * 0.25
        h = jnp.maximum(
            jax.lax.dot_general(stats8, w1_ref[...], (((1,), (0,)), ((), ())),
                                preferred_element_type=jnp.float32)
            + b1_ref[...], 0.0)
        calib = jax.lax.dot_general(h, w2_ref[...], (((1,), (0,)), ((), ())),
                                    preferred_element_type=jnp.float32) \
            + b2_ref[...]
        stats_ref[...] = stats8[:, 0:6]
        scores_ref[...] = base + calib


@jax.jit
def kernel(features, class_centers, representatives, W1, b1, W2, b2):
    nq = features.shape[0]
    reps = jnp.pad(representatives, ((0, REPS_PAD - N_REPS), (0, 0)))
    cents = jnp.pad(class_centers, ((0, 1024 - N_CENTERS), (0, 0)))
    w1 = jnp.pad(W1, ((0, 2), (0, 0)))
    b1r = b1.reshape(1, 16)
    b2r = b2.reshape(1, 1)

    stats, scores = pl.pallas_call(
        _kernel,
        grid=(N_TILES,),
        in_specs=[
            pl.BlockSpec((nq, 32), lambda i: (0, 0)),
            pl.BlockSpec((1024, 32), lambda i: (0, 0)),
            pl.BlockSpec((TILE, 32), lambda i: (i, 0)),
            pl.BlockSpec((8, 16), lambda i: (0, 0)),
            pl.BlockSpec((1, 16), lambda i: (0, 0)),
            pl.BlockSpec((16, 1), lambda i: (0, 0)),
            pl.BlockSpec((1, 1), lambda i: (0, 0)),
        ],
        out_specs=[
            pl.BlockSpec((nq, 6), lambda i: (0, 0)),
            pl.BlockSpec((nq, 1), lambda i: (0, 0)),
        ],
        out_shape=[
            jax.ShapeDtypeStruct((nq, 6), jnp.float32),
            jax.ShapeDtypeStruct((nq, 1), jnp.float32),
        ],
        scratch_shapes=[
            pltpu.VMEM((nq, 8), jnp.float32),
            pltpu.VMEM((nq, 1), jnp.float32),
        ],
    )(features.astype(jnp.float32), cents, reps, w1, b1r, W2, b2r)

    return stats, scores[:, 0]


# speedup vs baseline: 109.3208x; 1.2579x over previous
import functools

import jax
import jax.numpy as jnp
from jax import lax
from jax.experimental import pallas as pl
from jax.experimental.pallas import tpu as pltpu
from jax.experimental.pallas import tpu_sc as plsc

NQ = 1024
N_REPS = 100000
N_CENTERS = 1000
TILE = 2048
N_TILES = 49
REPS_PAD = N_TILES * TILE
SLOT = 128
SPT = TILE // SLOT
N_SLOTS = N_TILES * SPT
N_CAND = 5 * NQ
NEG = -1e30

NC, NS = 2, 16
NW = NC * NS
BPW = N_CAND // NW


def _rownorm_scale(x, eps=1e-12):
    n = jnp.sqrt(jnp.sum(x * x, axis=1, keepdims=True))
    return x / jnp.maximum(n, eps)


def _bank_kernel(feat_ref, reps_ref, table_ref, sum_ref, idx_ref,
                 top_ref, id_ref):
    i = pl.program_id(0)

    @pl.when(i == 0)
    def _init():
        top_ref[...] = jnp.full_like(top_ref, NEG)
        id_ref[...] = jnp.zeros_like(id_ref)
        sum_ref[...] = jnp.zeros_like(sum_ref)

    nf = _rownorm_scale(feat_ref[...])
    reps = _rownorm_scale(reps_ref[...])

    lane_io = lax.broadcasted_iota(jnp.int32, (1, SLOT), 1)
    rsum = jnp.zeros((NQ, 1), jnp.float32)
    slot_vals = []
    for s in range(SPT):
        rs = reps[s * SLOT:(s + 1) * SLOT, :]
        sims = lax.dot_general(nf, rs, (((1,), (1,)), ((), ())),
                               preferred_element_type=jnp.float32)
        rsum += jnp.sum(sims, axis=1, keepdims=True)
        base = i * TILE + s * SLOT
        pen = jnp.where(lane_io + base >= N_REPS, NEG, 0.0)
        sims = sims + pen
        table_ref[pl.ds(s * NQ, NQ), :] = sims
        slot_vals.append(jnp.max(sims, axis=1, keepdims=True))
    sum_ref[...] += rsum

    svals = jnp.concatenate(slot_vals, axis=1)
    sids = lax.broadcasted_iota(jnp.int32, (NQ, SPT), 1) + i * SPT

    vals = jnp.concatenate([top_ref[...], svals], axis=1)
    ids = jnp.concatenate([id_ref[...], sids], axis=1)
    w = vals.shape[1]
    io = lax.broadcasted_iota(jnp.int32, (NQ, w), 1)
    new_vals, new_ids = [], []
    for _ in range(5):
        m = jnp.max(vals, axis=1, keepdims=True)
        first = jnp.min(jnp.where(vals == m, io, w), axis=1, keepdims=True)
        sel = io == first
        new_ids.append(jnp.sum(jnp.where(sel, ids, 0), axis=1, keepdims=True))
        new_vals.append(m)
        vals = jnp.where(sel, NEG, vals)
    padv = jnp.full((NQ, 1), NEG, jnp.float32)
    padi = jnp.zeros((NQ, 1), jnp.int32)
    top_ref[...] = jnp.concatenate(new_vals + [padv] * 3, axis=1)
    id_ref[...] = jnp.concatenate(new_ids + [padi] * 3, axis=1)

    @pl.when(i == N_TILES - 1)
    def _emit_idx():
        qio = lax.broadcasted_iota(jnp.int32, (NQ, 8), 0)
        idx_ref[...] = id_ref[...] * NQ + qio


@functools.cache
def _make_gather_kernel():
    mesh = plsc.VectorSubcoreMesh(core_axis_name="c", subcore_axis_name="s",
                                  num_cores=NC, num_subcores=NS)

    @functools.partial(
        pl.kernel,
        mesh=mesh,
        out_type=jax.ShapeDtypeStruct((N_CAND, SLOT), jnp.float32),
        scratch_types=[
            pltpu.VMEM((BPW,), jnp.int32),
            pltpu.VMEM((BPW, SLOT), jnp.float32),
            pltpu.SemaphoreType.DMA,
        ],
    )
    def _gather_kernel(table_hbm, idx_hbm, out_hbm, idx_v, rows_v, sem):
        wid = lax.axis_index("s") * NC + lax.axis_index("c")
        base = wid * BPW
        pltpu.sync_copy(idx_hbm.at[pl.ds(base, BPW)], idx_v)
        pltpu.async_copy(table_hbm.at[idx_v], rows_v, sem).wait()
        pltpu.sync_copy(rows_v, out_hbm.at[pl.ds(base, BPW)])

    return _gather_kernel


def _final_kernel(feat_ref, cent_ref, cand_ref, sum_ref, w1_ref, b1_ref,
                  w2_ref, b2_ref, stats_ref, scores_ref):
    feats = feat_ref[...]
    nf = _rownorm_scale(feats)

    cents = _rownorm_scale(cent_ref[...])
    csims = lax.dot_general(nf, cents, (((1,), (1,)), ((), ())),
                            preferred_element_type=jnp.float32)
    cio = lax.broadcasted_iota(jnp.int32, csims.shape, 1)
    max_center = jnp.max(jnp.where(cio < N_CENTERS, csims, NEG),
                         axis=1, keepdims=True)
    mean_center = jnp.sum(csims, axis=1, keepdims=True) / N_CENTERS

    x = cand_ref[...]
    w = x.shape[1]
    io = lax.broadcasted_iota(jnp.int32, (NQ, w), 1)
    tops = []
    for _ in range(5):
        m = jnp.max(x, axis=1, keepdims=True)
        first = jnp.min(jnp.where(x == m, io, w), axis=1, keepdims=True)
        x = jnp.where(io == first, NEG, x)
        tops.append(m)
    rep_max = tops[0]
    rep_mean = (tops[0] + tops[1] + tops[2] + tops[3] + tops[4]) * 0.2
    rep_gmean = sum_ref[...] / N_REPS
    feat_norm = jnp.sqrt(jnp.sum(feats * feats, axis=1, keepdims=True))

    zero = jnp.zeros_like(rep_max)
    stats8 = jnp.concatenate(
        [max_center, mean_center, rep_max, rep_mean, rep_gmean,
         feat_norm, zero, zero], axis=1)

    base = (max_center + mean_center + rep_max + rep_mean) * 0.25
    h = jnp.maximum(
        lax.dot_general(stats8, w1_ref[...], (((1,), (0,)), ((), ())),
                        preferred_element_type=jnp.float32) + b1_ref[...],
        0.0)
    calib = lax.dot_general(h, w2_ref[...], (((1,), (0,)), ((), ())),
                            preferred_element_type=jnp.float32) + b2_ref[...]
    stats_ref[...] = stats8[:, 0:6]
    scores_ref[...] = base + calib


@jax.jit
def kernel(features, class_centers, representatives, W1, b1, W2, b2):
    feats = features.astype(jnp.float32)
    reps = jnp.pad(representatives, ((0, REPS_PAD - N_REPS), (0, 0)))
    cents = jnp.pad(class_centers, ((0, 1024 - N_CENTERS), (0, 0)))
    w1 = jnp.pad(W1, ((0, 2), (0, 0)))
    b1r = b1.reshape(1, 16)
    b2r = b2.reshape(1, 1)

    table, rsum, idx8 = pl.pallas_call(
        _bank_kernel,
        grid=(N_TILES,),
        in_specs=[
            pl.BlockSpec((NQ, 32), lambda i: (0, 0)),
            pl.BlockSpec((TILE, 32), lambda i: (i, 0)),
        ],
        out_specs=[
            pl.BlockSpec((SPT * NQ, SLOT), lambda i: (i, 0)),
            pl.BlockSpec((NQ, 1), lambda i: (0, 0)),
            pl.BlockSpec((NQ, 8), lambda i: (0, 0)),
        ],
        out_shape=[
            jax.ShapeDtypeStruct((N_SLOTS * NQ, SLOT), jnp.float32),
            jax.ShapeDtypeStruct((NQ, 1), jnp.float32),
            jax.ShapeDtypeStruct((NQ, 8), jnp.int32),
        ],
        scratch_shapes=[
            pltpu.VMEM((NQ, 8), jnp.float32),
            pltpu.VMEM((NQ, 8), jnp.int32),
        ],
    )(feats, reps)

    idx_flat = idx8[:, :5].reshape(N_CAND)
    gathered = _make_gather_kernel()(table, idx_flat)
    cand = gathered.reshape(NQ, 5 * SLOT)

    stats, scores = pl.pallas_call(
        _final_kernel,
        in_specs=[
            pl.BlockSpec((NQ, 32), lambda: (0, 0)),
            pl.BlockSpec((1024, 32), lambda: (0, 0)),
            pl.BlockSpec((NQ, 5 * SLOT), lambda: (0, 0)),
            pl.BlockSpec((NQ, 1), lambda: (0, 0)),
            pl.BlockSpec((8, 16), lambda: (0, 0)),
            pl.BlockSpec((1, 16), lambda: (0, 0)),
            pl.BlockSpec((16, 1), lambda: (0, 0)),
            pl.BlockSpec((1, 1), lambda: (0, 0)),
        ],
        out_specs=[
            pl.BlockSpec((NQ, 6), lambda: (0, 0)),
            pl.BlockSpec((NQ, 1), lambda: (0, 0)),
        ],
        out_shape=[
            jax.ShapeDtypeStruct((NQ, 6), jnp.float32),
            jax.ShapeDtypeStruct((NQ, 1), jnp.float32),
        ],
    )(feats, cents, cand, rsum, w1, b1r, W2, b2r)

    return stats, scores[:, 0]


# speedup vs baseline: 214.9830x; 1.9665x over previous
import functools

import jax
import jax.numpy as jnp
from jax import lax
from jax.experimental import pallas as pl
from jax.experimental.pallas import tpu as pltpu
from jax.experimental.pallas import tpu_sc as plsc

NQ = 1024
N_REPS = 100000
N_CENTERS = 1000
TILE = 2048
N_TILES = 49
REPS_PAD = N_TILES * TILE
SLOT = 128
SPT = TILE // SLOT
N_SLOTS = N_TILES * SPT
N_CAND = 5 * NQ
NEG = -1e30

NC, NS = 2, 16
NW = NC * NS
BPW = N_CAND // NW


def _rownorm_scale(x, eps=1e-12):
    n = jnp.sqrt(jnp.sum(x * x, axis=1, keepdims=True))
    return x / jnp.maximum(n, eps)


def _bank_kernel(feat_ref, reps_ref, table_ref, smax_ref, sum_ref,
                 nf_ref, acc_ref):
    i = pl.program_id(0)

    @pl.when(i == 0)
    def _init():
        nf_ref[...] = _rownorm_scale(feat_ref[...])
        acc_ref[...] = jnp.zeros_like(acc_ref)

    nf = nf_ref[...]
    reps = _rownorm_scale(reps_ref[...])
    sims = lax.dot_general(nf, reps, (((1,), (1,)), ((), ())),
                           preferred_element_type=jnp.float32)

    acc = acc_ref[...]
    for s in range(SPT):
        acc += sims[:, s * SLOT:(s + 1) * SLOT]
    acc_ref[...] = acc

    lane_io = lax.broadcasted_iota(jnp.int32, (1, TILE), 1)
    pen = jnp.where(lane_io + i * TILE >= N_REPS, NEG, 0.0)
    sims = sims + pen

    for s in range(SPT):
        sl = sims[:, s * SLOT:(s + 1) * SLOT]
        table_ref[pl.ds(s * NQ, NQ), :] = sl
        smax_ref[0, :, s:s + 1] = jnp.max(sl, axis=1, keepdims=True)

    @pl.when(i == N_TILES - 1)
    def _emit_sum():
        sum_ref[...] = jnp.sum(acc_ref[...], axis=1, keepdims=True)


def _slotsel_kernel(smax_ref, idx_ref):
    x = smax_ref[...]
    w = x.shape[1]
    io = lax.broadcasted_iota(jnp.int32, (NQ, w), 1)
    qio = lax.broadcasted_iota(jnp.int32, (NQ, 1), 0)
    for k in range(5):
        m = jnp.max(x, axis=1, keepdims=True)
        first = jnp.min(jnp.where(x == m, io, w), axis=1, keepdims=True)
        x = jnp.where(io == first, NEG, x)
        idx_ref[:, k:k + 1] = first * NQ + qio
    idx_ref[:, 5:8] = jnp.zeros((NQ, 3), jnp.int32)


@functools.cache
def _make_gather_kernel():
    mesh = plsc.VectorSubcoreMesh(core_axis_name="c", subcore_axis_name="s",
                                  num_cores=NC, num_subcores=NS)

    @functools.partial(
        pl.kernel,
        mesh=mesh,
        out_type=jax.ShapeDtypeStruct((N_CAND, SLOT), jnp.float32),
        scratch_types=[
            pltpu.VMEM((BPW,), jnp.int32),
            pltpu.VMEM((BPW, SLOT), jnp.float32),
            pltpu.SemaphoreType.DMA,
        ],
    )
    def _gather_kernel(table_hbm, idx_hbm, out_hbm, idx_v, rows_v, sem):
        wid = lax.axis_index("s") * NC + lax.axis_index("c")
        base = wid * BPW
        pltpu.sync_copy(idx_hbm.at[pl.ds(base, BPW)], idx_v)
        pltpu.async_copy(table_hbm.at[idx_v], rows_v, sem).wait()
        pltpu.sync_copy(rows_v, out_hbm.at[pl.ds(base, BPW)])

    return _gather_kernel


def _final_kernel(feat_ref, cent_ref, cand_ref, sum_ref, w1_ref, b1_ref,
                  w2_ref, b2_ref, stats_ref, scores_ref):
    feats = feat_ref[...]
    nf = _rownorm_scale(feats)

    cents = _rownorm_scale(cent_ref[...])
    csims = lax.dot_general(nf, cents, (((1,), (1,)), ((), ())),
                            preferred_element_type=jnp.float32)
    cio = lax.broadcasted_iota(jnp.int32, csims.shape, 1)
    max_center = jnp.max(jnp.where(cio < N_CENTERS, csims, NEG),
                         axis=1, keepdims=True)
    mean_center = jnp.sum(csims, axis=1, keepdims=True) / N_CENTERS

    x = cand_ref[...]
    w = x.shape[1]
    io = lax.broadcasted_iota(jnp.int32, (NQ, w), 1)
    tops = []
    for _ in range(5):
        m = jnp.max(x, axis=1, keepdims=True)
        first = jnp.min(jnp.where(x == m, io, w), axis=1, keepdims=True)
        x = jnp.where(io == first, NEG, x)
        tops.append(m)
    rep_max = tops[0]
    rep_mean = (tops[0] + tops[1] + tops[2] + tops[3] + tops[4]) * 0.2
    rep_gmean = sum_ref[...] / N_REPS
    feat_norm = jnp.sqrt(jnp.sum(feats * feats, axis=1, keepdims=True))

    zero = jnp.zeros_like(rep_max)
    stats8 = jnp.concatenate(
        [max_center, mean_center, rep_max, rep_mean, rep_gmean,
         feat_norm, zero, zero], axis=1)

    base = (max_center + mean_center + rep_max + rep_mean) * 0.25
    h = jnp.maximum(
        lax.dot_general(stats8, w1_ref[...], (((1,), (0,)), ((), ())),
                        preferred_element_type=jnp.float32) + b1_ref[...],
        0.0)
    calib = lax.dot_general(h, w2_ref[...], (((1,), (0,)), ((), ())),
                            preferred_element_type=jnp.float32) + b2_ref[...]
    stats_ref[...] = stats8[:, 0:6]
    scores_ref[...] = base + calib


@jax.jit
def kernel(features, class_centers, representatives, W1, b1, W2, b2):
    feats = features.astype(jnp.float32)
    reps = jnp.pad(representatives, ((0, REPS_PAD - N_REPS), (0, 0)))
    cents = jnp.pad(class_centers, ((0, 1024 - N_CENTERS), (0, 0)))
    w1 = jnp.pad(W1, ((0, 2), (0, 0)))
    b1r = b1.reshape(1, 16)
    b2r = b2.reshape(1, 1)

    table, smax3, rsum = pl.pallas_call(
        _bank_kernel,
        grid=(N_TILES,),
        in_specs=[
            pl.BlockSpec((NQ, 32), lambda i: (0, 0)),
            pl.BlockSpec((TILE, 32), lambda i: (i, 0)),
        ],
        out_specs=[
            pl.BlockSpec((SPT * NQ, SLOT), lambda i: (i, 0)),
            pl.BlockSpec((1, NQ, SPT), lambda i: (i, 0, 0)),
            pl.BlockSpec((NQ, 1), lambda i: (0, 0)),
        ],
        out_shape=[
            jax.ShapeDtypeStruct((N_SLOTS * NQ, SLOT), jnp.float32),
            jax.ShapeDtypeStruct((N_TILES, NQ, SPT), jnp.float32),
            jax.ShapeDtypeStruct((NQ, 1), jnp.float32),
        ],
        scratch_shapes=[
            pltpu.VMEM((NQ, 32), jnp.float32),
            pltpu.VMEM((NQ, SLOT), jnp.float32),
        ],
    )(feats, reps)

    smax = smax3.transpose(1, 0, 2).reshape(NQ, N_SLOTS)
    idx8 = pl.pallas_call(
        _slotsel_kernel,
        in_specs=[pl.BlockSpec((NQ, N_SLOTS), lambda: (0, 0))],
        out_specs=pl.BlockSpec((NQ, 8), lambda: (0, 0)),
        out_shape=jax.ShapeDtypeStruct((NQ, 8), jnp.int32),
    )(smax)

    idx_flat = idx8[:, :5].reshape(N_CAND)
    gathered = _make_gather_kernel()(table, idx_flat)
    cand = gathered.reshape(NQ, 5 * SLOT)

    stats, scores = pl.pallas_call(
        _final_kernel,
        in_specs=[
            pl.BlockSpec((NQ, 32), lambda: (0, 0)),
            pl.BlockSpec((1024, 32), lambda: (0, 0)),
            pl.BlockSpec((NQ, 5 * SLOT), lambda: (0, 0)),
            pl.BlockSpec((NQ, 1), lambda: (0, 0)),
            pl.BlockSpec((8, 16), lambda: (0, 0)),
            pl.BlockSpec((1, 16), lambda: (0, 0)),
            pl.BlockSpec((16, 1), lambda: (0, 0)),
            pl.BlockSpec((1, 1), lambda: (0, 0)),
        ],
        out_specs=[
            pl.BlockSpec((NQ, 6), lambda: (0, 0)),
            pl.BlockSpec((NQ, 1), lambda: (0, 0)),
        ],
        out_shape=[
            jax.ShapeDtypeStruct((NQ, 6), jnp.float32),
            jax.ShapeDtypeStruct((NQ, 1), jnp.float32),
        ],
    )(feats, cents, cand, rsum, w1, b1r, W2, b2r)

    return stats, scores[:, 0]


# speedup vs baseline: 235.6405x; 1.0961x over previous
import functools

import jax
import jax.numpy as jnp
from jax import lax
from jax.experimental import pallas as pl
from jax.experimental.pallas import tpu as pltpu
from jax.experimental.pallas import tpu_sc as plsc

NQ = 1024
N_REPS = 100000
N_CENTERS = 1000
TILE = 2048
N_TILES = 49
REPS_PAD = N_TILES * TILE
SLOT = 128
SPT = TILE // SLOT
N_SLOTS = N_TILES * SPT
N_CAND = 5 * NQ
NEG = -1e30
_FIRST_PAD_SLOT = (N_REPS - (N_TILES - 1) * TILE) // SLOT

NC, NS = 2, 16
NW = NC * NS
BPW = N_CAND // NW


def _rownorm_scale(x, eps2=1e-24):
    n2 = jnp.sum(x * x, axis=1, keepdims=True)
    return x * lax.rsqrt(jnp.maximum(n2, eps2))


def _bank_kernel(feat_ref, reps_ref, tail_ref, table_ref, smax_ref, sum_ref,
                 nf_ref, acc_ref):
    i = pl.program_id(0)

    @pl.when(i == 0)
    def _init():
        nf_ref[...] = _rownorm_scale(feat_ref[...])
        acc_ref[...] = jnp.zeros_like(acc_ref)

    nf = nf_ref[...]
    raw = jnp.where(i == N_TILES - 1, tail_ref[...], reps_ref[...])
    reps = _rownorm_scale(raw)
    sims = lax.dot_general(nf, reps, (((1,), (1,)), ((), ())),
                           preferred_element_type=jnp.float32)

    acc = acc_ref[...]
    for s in range(SPT):
        acc += sims[:, s * SLOT:(s + 1) * SLOT]
    acc_ref[...] = acc

    for s in range(SPT):
        sl = sims[:, s * SLOT:(s + 1) * SLOT]
        table_ref[pl.ds(s * NQ, NQ), :] = sl
        smax_ref[0, :, s:s + 1] = jnp.max(sl, axis=1, keepdims=True)

    @pl.when(i == N_TILES - 1)
    def _fix_tail_and_emit_sum():
        lane_io = lax.broadcasted_iota(jnp.int32, (1, SLOT), 1)
        for s in range(_FIRST_PAD_SLOT, SPT):
            base = (N_TILES - 1) * TILE + s * SLOT
            pen = jnp.where(lane_io + base >= N_REPS, NEG, 0.0)
            sl = sims[:, s * SLOT:(s + 1) * SLOT] + pen
            table_ref[pl.ds(s * NQ, NQ), :] = sl
            smax_ref[0, :, s:s + 1] = jnp.max(sl, axis=1, keepdims=True)
        sum_ref[...] = jnp.sum(acc_ref[...], axis=1, keepdims=True)


def _slotsel_kernel(smax_ref, idx_ref):
    x = smax_ref[...]
    w = x.shape[1]
    io = lax.broadcasted_iota(jnp.int32, (NQ, w), 1)
    qio = lax.broadcasted_iota(jnp.int32, (NQ, 1), 0)
    for k in range(5):
        m = jnp.max(x, axis=1, keepdims=True)
        first = jnp.min(jnp.where(x == m, io, w), axis=1, keepdims=True)
        x = jnp.where(io == first, NEG, x)
        idx_ref[:, k:k + 1] = first * NQ + qio
    idx_ref[:, 5:8] = jnp.zeros((NQ, 3), jnp.int32)


@functools.cache
def _make_gather_kernel():
    mesh = plsc.VectorSubcoreMesh(core_axis_name="c", subcore_axis_name="s",
                                  num_cores=NC, num_subcores=NS)

    @functools.partial(
        pl.kernel,
        mesh=mesh,
        out_type=jax.ShapeDtypeStruct((N_CAND, SLOT), jnp.float32),
        scratch_types=[
            pltpu.VMEM((BPW,), jnp.int32),
            pltpu.VMEM((BPW, SLOT), jnp.float32),
            pltpu.SemaphoreType.DMA,
        ],
    )
    def _gather_kernel(table_hbm, idx_hbm, out_hbm, idx_v, rows_v, sem):
        wid = lax.axis_index("s") * NC + lax.axis_index("c")
        base = wid * BPW
        pltpu.sync_copy(idx_hbm.at[pl.ds(base, BPW)], idx_v)
        pltpu.async_copy(table_hbm.at[idx_v], rows_v, sem).wait()
        pltpu.sync_copy(rows_v, out_hbm.at[pl.ds(base, BPW)])

    return _gather_kernel


def _final_kernel(feat_ref, cent_ref, cand_ref, sum_ref, w1_ref, b1_ref,
                  w2_ref, b2_ref, stats_ref, scores_ref):
    feats = feat_ref[...]
    nf = _rownorm_scale(feats)

    cents = _rownorm_scale(cent_ref[...])
    csims = lax.dot_general(nf, cents, (((1,), (1,)), ((), ())),
                            preferred_element_type=jnp.float32)
    cio = lax.broadcasted_iota(jnp.int32, csims.shape, 1)
    max_center = jnp.max(jnp.where(cio < N_CENTERS, csims, NEG),
                         axis=1, keepdims=True)
    mean_center = jnp.sum(csims, axis=1, keepdims=True) / N_CENTERS

    x5 = cand_ref[...]
    x = jnp.concatenate([x5[k] for k in range(5)], axis=1)
    w = x.shape[1]
    io = lax.broadcasted_iota(jnp.int32, (NQ, w), 1)
    tops = []
    for _ in range(5):
        m = jnp.max(x, axis=1, keepdims=True)
        first = jnp.min(jnp.where(x == m, io, w), axis=1, keepdims=True)
        x = jnp.where(io == first, NEG, x)
        tops.append(m)
    rep_max = tops[0]
    rep_mean = (tops[0] + tops[1] + tops[2] + tops[3] + tops[4]) * 0.2
    rep_gmean = sum_ref[...] / N_REPS
    feat_norm = jnp.sqrt(jnp.sum(feats * feats, axis=1, keepdims=True))

    zero = jnp.zeros_like(rep_max)
    stats8 = jnp.concatenate(
        [max_center, mean_center, rep_max, rep_mean, rep_gmean,
         feat_norm, zero, zero], axis=1)

    base = (max_center + mean_center + rep_max + rep_mean) * 0.25
    h = jnp.maximum(
        lax.dot_general(stats8, w1_ref[...], (((1,), (0,)), ((), ())),
                        preferred_element_type=jnp.float32) + b1_ref[...],
        0.0)
    calib = lax.dot_general(h, w2_ref[...], (((1,), (0,)), ((), ())),
                            preferred_element_type=jnp.float32) + b2_ref[...]
    stats_ref[...] = stats8[:, 0:6]
    scores_ref[...] = base + calib


@jax.jit
def kernel(features, class_centers, representatives, W1, b1, W2, b2):
    feats = features.astype(jnp.float32)
    tail = jnp.pad(representatives[(N_TILES - 1) * TILE:],
                   ((0, REPS_PAD - N_REPS), (0, 0)))
    cents = jnp.pad(class_centers, ((0, 1024 - N_CENTERS), (0, 0)))
    w1 = jnp.pad(W1, ((0, 2), (0, 0)))
    b1r = b1.reshape(1, 16)
    b2r = b2.reshape(1, 1)

    table, smax3, rsum = pl.pallas_call(
        _bank_kernel,
        grid=(N_TILES,),
        in_specs=[
            pl.BlockSpec((NQ, 32), lambda i: (0, 0)),
            pl.BlockSpec((TILE, 32), lambda i: (i, 0)),
            pl.BlockSpec((TILE, 32), lambda i: (0, 0)),
        ],
        out_specs=[
            pl.BlockSpec((SPT * NQ, SLOT), lambda i: (i, 0)),
            pl.BlockSpec((1, NQ, SPT), lambda i: (i, 0, 0)),
            pl.BlockSpec((NQ, 1), lambda i: (0, 0)),
        ],
        out_shape=[
            jax.ShapeDtypeStruct((N_SLOTS * NQ, SLOT), jnp.float32),
            jax.ShapeDtypeStruct((N_TILES, NQ, SPT), jnp.float32),
            jax.ShapeDtypeStruct((NQ, 1), jnp.float32),
        ],
        scratch_shapes=[
            pltpu.VMEM((NQ, 32), jnp.float32),
            pltpu.VMEM((NQ, SLOT), jnp.float32),
        ],
    )(feats, representatives, tail)

    smax = smax3.transpose(1, 0, 2).reshape(NQ, N_SLOTS)
    idx8 = pl.pallas_call(
        _slotsel_kernel,
        in_specs=[pl.BlockSpec((NQ, N_SLOTS), lambda: (0, 0))],
        out_specs=pl.BlockSpec((NQ, 8), lambda: (0, 0)),
        out_shape=jax.ShapeDtypeStruct((NQ, 8), jnp.int32),
    )(smax)

    idx_flat = idx8[:, :5].T.reshape(N_CAND)
    gathered = _make_gather_kernel()(table, idx_flat)
    cand = gathered.reshape(5, NQ, SLOT)

    stats, scores = pl.pallas_call(
        _final_kernel,
        in_specs=[
            pl.BlockSpec((NQ, 32), lambda: (0, 0)),
            pl.BlockSpec((1024, 32), lambda: (0, 0)),
            pl.BlockSpec((5, NQ, SLOT), lambda: (0, 0, 0)),
            pl.BlockSpec((NQ, 1), lambda: (0, 0)),
            pl.BlockSpec((8, 16), lambda: (0, 0)),
            pl.BlockSpec((1, 16), lambda: (0, 0)),
            pl.BlockSpec((16, 1), lambda: (0, 0)),
            pl.BlockSpec((1, 1), lambda: (0, 0)),
        ],
        out_specs=[
            pl.BlockSpec((NQ, 6), lambda: (0, 0)),
            pl.BlockSpec((NQ, 1), lambda: (0, 0)),
        ],
        out_shape=[
            jax.ShapeDtypeStruct((NQ, 6), jnp.float32),
            jax.ShapeDtypeStruct((NQ, 1), jnp.float32),
        ],
    )(feats, cents, cand, rsum, w1, b1r, W2, b2r)

    return stats, scores[:, 0]
